# Initial kernel scaffold; baseline (speedup 1.0000x reference)
import jax
import jax.numpy as jnp
from jax.experimental import pallas as pl


def kernel(x, Wr, W1, W2):
    raise NotImplementedError("write your pallas kernel here")


# speedup vs baseline: 601.9206x; 601.9206x over previous
import functools

import jax
import jax.numpy as jnp
from jax import lax
from jax.experimental import pallas as pl
from jax.experimental.pallas import tpu as pltpu
from jax.experimental.pallas import tpu_sc as plsc

B, S, D, DFF = 4, 4096, 2048, 8192
CAP = S // 2
ZROW = B * CAP
NC, NS = 2, 16
WPB = NS // 2


def _router_body(x_ref, wr_ref, rw2_ref):
    xb = x_ref[...]
    z = jnp.dot(xb, wr_ref[...], preferred_element_type=jnp.float32)
    rw2_ref[...] = jax.nn.sigmoid(z[:, 0]).reshape(1, 1, -1)


def _router(x, Wr):
    BMR = 1024
    xf = x.reshape(B * S, D)
    grid = (B * S // BMR,)
    rw2 = pl.pallas_call(
        _router_body,
        grid=grid,
        in_specs=[
            pl.BlockSpec((BMR, D), lambda i: (i, 0)),
            pl.BlockSpec((D, 1), lambda i: (0, 0)),
        ],
        out_specs=pl.BlockSpec((1, 1, BMR), lambda i: (i, 0, 0)),
        out_shape=jax.ShapeDtypeStruct((B * S // BMR, 1, BMR), jnp.float32),
    )(xf, Wr)
    return rw2.reshape(B, 1, S)


_BT = 512


def _rank_body(rw2_ref, rank_ref, g_ref, loss_ref):
    bstep = pl.program_id(0)
    v = rw2_ref[0, 0, :]
    tri = (lax.broadcasted_iota(jnp.int32, (_BT, _BT), 1)
           < lax.broadcasted_iota(jnp.int32, (_BT, _BT), 0))
    total = jnp.zeros((1, 1), jnp.float32)
    segs = []
    for j in range(S // _BT):
        vt = v[j * _BT:(j + 1) * _BT]
        cnt = jnp.zeros((_BT,), jnp.int32)
        for st in range(S // _BT):
            vs = v[st * _BT:(st + 1) * _BT]
            if st < j:
                m = vs[None, :] >= vt[:, None]
            elif st > j:
                m = vs[None, :] > vt[:, None]
            else:
                m = (vs[None, :] > vt[:, None]) | (
                    (vs[None, :] == vt[:, None]) & tri)
            cnt = cnt + jnp.sum(m.astype(jnp.int32), axis=1)
        rank_ref[0, 0, pl.ds(j * _BT, _BT)] = cnt
        segs.append(cnt)
        sel = cnt < CAP
        p = jnp.clip(vt, 1e-12, 1.0 - 1e-12)
        term = jnp.where(sel, -jnp.log(p), -jnp.log(1.0 - p))
        total = total + jnp.sum(term)

    r = jnp.concatenate(segs)
    sel = (r < CAP).astype(jnp.int32)
    pos = sel
    k = 1
    while k < S:
        pos = pos + jnp.concatenate(
            [jnp.zeros((k,), jnp.int32), pos[: S - k]])
        k *= 2
    pos = pos - sel
    g_ref[0, 0, :] = jnp.where(r < CAP, bstep * CAP + pos, ZROW)

    @pl.when(bstep == 0)
    def _():
        loss_ref[...] = jnp.zeros((1, 1), jnp.float32)

    acc = loss_ref[...] + total
    loss_ref[...] = jnp.where(bstep == B - 1, acc / (B * S), acc)


def _ranks_loss(rw2_3d):
    grid = (B,)
    rank, g, loss = pl.pallas_call(
        _rank_body,
        grid=grid,
        in_specs=[pl.BlockSpec((1, 1, S), lambda b: (b, 0, 0))],
        out_specs=[
            pl.BlockSpec((1, 1, S), lambda b: (b, 0, 0)),
            pl.BlockSpec((1, 1, S), lambda b: (b, 0, 0)),
            pl.BlockSpec((1, 1), lambda b: (0, 0)),
        ],
        out_shape=[
            jax.ShapeDtypeStruct((B, 1, S), jnp.int32),
            jax.ShapeDtypeStruct((B, 1, S), jnp.int32),
            jax.ShapeDtypeStruct((1, 1), jnp.float32),
        ],
    )(rw2_3d)
    return rank, g.reshape(B, S), loss[0, 0]


_GCH = 16
_NCH = CAP // WPB // _GCH


def _route_gather_body(rw2_hbm, rank_hbm, gmap_hbm, x_hbm, wts_hbm, modx_hbm,
                       rw2_v, rank_v, sidx_v, wt_v, g_v, idx_v,
                       rows_a, rows_b, sidx_sh, sem_a, sem_b):
    c = lax.axis_index("c")
    s = lax.axis_index("s")

    @pl.when(s < 2)
    def _compact():
        b = c * 2 + s
        pltpu.sync_copy(rank_hbm.at[b], rank_v)
        pltpu.sync_copy(rw2_hbm.at[b], rw2_v)
        pltpu.sync_copy(gmap_hbm.at[b], g_v)
        base = b * CAP

        def step(i, _):
            rv = rank_v[pl.ds(i * 16, 16)]
            gv = g_v[pl.ds(i * 16, 16)]
            sel = gv != ZROW
            pos = gv - base
            toks = lax.iota(jnp.int32, 16) + i * 16
            plsc.store_scatter(sidx_v, [pos], toks, mask=sel)
            wvals = plsc.load_gather(rw2_v, [rv])
            plsc.store_scatter(wt_v, [pos], wvals, mask=sel)
            return 0

        lax.fori_loop(0, S // 16, step, 0)
        pltpu.sync_copy(wt_v, wts_hbm.at[b])
        pltpu.sync_copy(sidx_v, sidx_sh.at[s])

    plsc.subcore_barrier()

    half = s // WPB
    b = c * 2 + half
    w8 = s % WPB
    slot0 = w8 * (CAP // WPB)
    pltpu.sync_copy(sidx_sh.at[half, pl.ds(slot0, CAP // WPB)], idx_v)

    bufs = (rows_a, rows_b)
    sems = (sem_a, sem_b)
    xb = x_hbm.at[b]

    def gather_pair(k, _):
        cps = []
        for q in range(2):
            ch = k * 2 + q
            cps.append(pltpu.async_copy(
                xb.at[idx_v.at[pl.ds(ch * _GCH, _GCH)]], bufs[q], sems[q]))
        for q in range(2):
            ch = k * 2 + q
            cps[q].wait()
            row0 = b * CAP + slot0 + ch * _GCH
            pltpu.sync_copy(bufs[q], modx_hbm.at[pl.ds(row0, _GCH)])
        return 0

    lax.fori_loop(0, _NCH // 2, gather_pair, 0)


def _route_gather(rw2, rank, gmap, x):
    mesh = plsc.VectorSubcoreMesh(core_axis_name="c", subcore_axis_name="s",
                                  num_cores=NC, num_subcores=NS)
    f = pl.kernel(
        _route_gather_body,
        out_type=[
            jax.ShapeDtypeStruct((B, CAP), jnp.float32),
            jax.ShapeDtypeStruct((B * CAP, D), jnp.float32),
        ],
        mesh=mesh,
        scratch_types=[
            pltpu.VMEM((S,), jnp.float32),
            pltpu.VMEM((S,), jnp.int32),
            pltpu.VMEM((CAP,), jnp.int32),
            pltpu.VMEM((CAP,), jnp.float32),
            pltpu.VMEM((S,), jnp.int32),
            pltpu.VMEM((CAP // WPB,), jnp.int32),
            pltpu.VMEM((_GCH, D), jnp.float32),
            pltpu.VMEM((_GCH, D), jnp.float32),
            pltpu.VMEM_SHARED((2, CAP), jnp.int32),
            pltpu.SemaphoreType.DMA,
            pltpu.SemaphoreType.DMA,
        ],
        compiler_params=pltpu.CompilerParams(needs_layout_passes=False),
    )
    return f(rw2, rank, gmap, x)


_BM = 1024
_BK = 1024


def _ffn_body(x_ref, w1_ref, w2_ref, wt_ref, out_ref, acc_ref):
    cch = pl.program_id(1)

    @pl.when(cch == 0)
    def _():
        acc_ref[...] = jnp.zeros_like(acc_ref)

    h = jax.nn.gelu(
        jnp.dot(x_ref[...].astype(jnp.bfloat16), w1_ref[...],
                preferred_element_type=jnp.float32))
    acc_ref[...] += jnp.dot(
        h.astype(jnp.bfloat16), w2_ref[...], preferred_element_type=jnp.float32)

    @pl.when(cch == DFF // _BK - 1)
    def _():
        wt = wt_ref[0, 0, :]
        out_ref[...] = acc_ref[...] * wt[:, None]


def _ffn(modx, W1, W2, wts):
    w1b = W1.astype(jnp.bfloat16)
    w2b = W2.astype(jnp.bfloat16)
    wt2 = wts.reshape(B * CAP // _BM, 1, _BM)
    grid = (B * CAP // _BM, DFF // _BK)
    out = pl.pallas_call(
        _ffn_body,
        grid=grid,
        in_specs=[
            pl.BlockSpec((_BM, D), lambda m, c: (m, 0)),
            pl.BlockSpec((D, _BK), lambda m, c: (0, c)),
            pl.BlockSpec((_BK, D), lambda m, c: (c, 0)),
            pl.BlockSpec((1, 1, _BM), lambda m, c: (m, 0, 0)),
        ],
        out_specs=pl.BlockSpec((_BM, D), lambda m, c: (m, 0)),
        out_shape=jax.ShapeDtypeStruct((B * CAP + 8, D), jnp.float32),
        scratch_shapes=[pltpu.VMEM((_BM, D), jnp.float32)],
        compiler_params=pltpu.CompilerParams(
            dimension_semantics=("parallel", "arbitrary"),
            vmem_limit_bytes=63 * 1024 * 1024),
    )(modx, w1b, w2b, wt2)
    return out


_OCH = 16
_TPW = S // WPB


def _out_gather_body(gmap_hbm, modout_hbm, out_hbm,
                     g_v, zrow_v, rows_a, rows_b, sem_a, sem_b):
    c = lax.axis_index("c")
    s = lax.axis_index("s")

    @pl.when(s == 0)
    def _zero_row():
        def zb(i, _):
            zrow_v[pl.ds(i * 16, 16)] = jnp.zeros((16,), jnp.float32)
            return 0

        lax.fori_loop(0, D // 16, zb, 0)
        pltpu.sync_copy(zrow_v, modout_hbm.at[ZROW])

    plsc.subcore_barrier()

    half = s // WPB
    b = c * 2 + half
    t0 = (s % WPB) * _TPW
    pltpu.sync_copy(gmap_hbm.at[b, pl.ds(t0, _TPW)], g_v)

    bufs = (rows_a, rows_b)
    sems = (sem_a, sem_b)
    ob = out_hbm.at[b]

    def pair(k, _):
        cps = []
        for q in range(2):
            ch = k * 2 + q
            cps.append(pltpu.async_copy(
                modout_hbm.at[g_v.at[pl.ds(ch * _OCH, _OCH)]], bufs[q], sems[q]))
        for q in range(2):
            ch = k * 2 + q
            cps[q].wait()
            pltpu.sync_copy(bufs[q], ob.at[pl.ds(t0 + ch * _OCH, _OCH)])
        return 0

    lax.fori_loop(0, _TPW // _OCH // 2, pair, 0)


def _out_gather(gmap, modout):
    mesh = plsc.VectorSubcoreMesh(core_axis_name="c", subcore_axis_name="s",
                                  num_cores=NC, num_subcores=NS)
    f = pl.kernel(
        _out_gather_body,
        out_type=jax.ShapeDtypeStruct((B, S, D), jnp.float32),
        mesh=mesh,
        scratch_types=[
            pltpu.VMEM((_TPW,), jnp.int32),
            pltpu.VMEM((D,), jnp.float32),
            pltpu.VMEM((_OCH, D), jnp.float32),
            pltpu.VMEM((_OCH, D), jnp.float32),
            pltpu.SemaphoreType.DMA,
            pltpu.SemaphoreType.DMA,
        ],
        compiler_params=pltpu.CompilerParams(needs_layout_passes=False),
    )
    return f(gmap, modout)


def kernel(x, Wr, W1, W2):
    rw2_3d = _router(x, Wr)
    rank_3d, gmap, mod_loss = _ranks_loss(rw2_3d)
    wts, modx = _route_gather(
        rw2_3d.reshape(B, S), rank_3d.reshape(B, S), gmap, x)
    modout = _ffn(modx, W1, W2, wts)
    out = _out_gather(gmap, modout)
    return out, mod_loss


# speedup vs baseline: 814.3550x; 1.3529x over previous
import functools

import jax
import jax.numpy as jnp
from jax import lax
from jax.experimental import pallas as pl
from jax.experimental.pallas import tpu as pltpu
from jax.experimental.pallas import tpu_sc as plsc

B, S, D, DFF = 4, 4096, 2048, 8192
CAP = S // 2
ZROW = B * CAP
NC, NS = 2, 16
WPB = NS // 2


def _router_body(x_ref, wr_ref, rw2_ref):
    xb = x_ref[...]
    z = jnp.dot(xb, wr_ref[...], preferred_element_type=jnp.float32)
    rw2_ref[...] = jax.nn.sigmoid(z[:, 0]).reshape(1, 1, -1)


def _router(x, Wr):
    BMR = 1024
    xf = x.reshape(B * S, D)
    grid = (B * S // BMR,)
    rw2 = pl.pallas_call(
        _router_body,
        grid=grid,
        in_specs=[
            pl.BlockSpec((BMR, D), lambda i: (i, 0)),
            pl.BlockSpec((D, 1), lambda i: (0, 0)),
        ],
        out_specs=pl.BlockSpec((1, 1, BMR), lambda i: (i, 0, 0)),
        out_shape=jax.ShapeDtypeStruct((B * S // BMR, 1, BMR), jnp.float32),
    )(xf, Wr)
    return rw2.reshape(B, 1, S)


_BT = 512


def _rank_body(rw2_ref, rank_ref, g_ref, loss_ref):
    bstep = pl.program_id(0)
    v = rw2_ref[0, 0, :]
    tri = (lax.broadcasted_iota(jnp.int32, (_BT, _BT), 1)
           < lax.broadcasted_iota(jnp.int32, (_BT, _BT), 0))
    total = jnp.zeros((1, 1), jnp.float32)
    segs = []
    for j in range(S // _BT):
        vt = v[j * _BT:(j + 1) * _BT]
        cnt = jnp.zeros((_BT,), jnp.int32)
        for st in range(S // _BT):
            vs = v[st * _BT:(st + 1) * _BT]
            if st < j:
                m = vs[None, :] >= vt[:, None]
            elif st > j:
                m = vs[None, :] > vt[:, None]
            else:
                m = (vs[None, :] > vt[:, None]) | (
                    (vs[None, :] == vt[:, None]) & tri)
            cnt = cnt + jnp.sum(m.astype(jnp.int32), axis=1)
        rank_ref[0, 0, pl.ds(j * _BT, _BT)] = cnt
        segs.append(cnt)
        sel = cnt < CAP
        p = jnp.clip(vt, 1e-12, 1.0 - 1e-12)
        term = jnp.where(sel, -jnp.log(p), -jnp.log(1.0 - p))
        total = total + jnp.sum(term)

    r = jnp.concatenate(segs)
    sel = (r < CAP).astype(jnp.int32)
    pos = sel
    k = 1
    while k < S:
        pos = pos + jnp.concatenate(
            [jnp.zeros((k,), jnp.int32), pos[: S - k]])
        k *= 2
    pos = pos - sel
    zrow = ZROW + (lax.broadcasted_iota(jnp.int32, (S,), 0) & 15)
    g_ref[0, 0, :] = jnp.where(r < CAP, bstep * CAP + pos, zrow)

    @pl.when(bstep == 0)
    def _():
        loss_ref[...] = jnp.zeros((1, 1), jnp.float32)

    acc = loss_ref[...] + total
    loss_ref[...] = jnp.where(bstep == B - 1, acc / (B * S), acc)


def _ranks_loss(rw2_3d):
    grid = (B,)
    rank, g, loss = pl.pallas_call(
        _rank_body,
        grid=grid,
        in_specs=[pl.BlockSpec((1, 1, S), lambda b: (b, 0, 0))],
        out_specs=[
            pl.BlockSpec((1, 1, S), lambda b: (b, 0, 0)),
            pl.BlockSpec((1, 1, S), lambda b: (b, 0, 0)),
            pl.BlockSpec((1, 1), lambda b: (0, 0)),
        ],
        out_shape=[
            jax.ShapeDtypeStruct((B, 1, S), jnp.int32),
            jax.ShapeDtypeStruct((B, 1, S), jnp.int32),
            jax.ShapeDtypeStruct((1, 1), jnp.float32),
        ],
    )(rw2_3d)
    return rank, g.reshape(B, S), loss[0, 0]


_GCH = 16
_NCH = CAP // WPB // _GCH


def _route_gather_body(rw2_hbm, rank_hbm, gmap_hbm, x_hbm, wts_hbm, modx_hbm,
                       rw2_v, rank_v, sidx_v, wt_v, g_v, idx_v,
                       rows_a, rows_b, sidx_sh, sem_a, sem_b,
                       wsem_a, wsem_b):
    c = lax.axis_index("c")
    s = lax.axis_index("s")

    @pl.when(s < 2)
    def _compact():
        b = c * 2 + s
        pltpu.sync_copy(rank_hbm.at[b], rank_v)
        pltpu.sync_copy(rw2_hbm.at[b], rw2_v)
        pltpu.sync_copy(gmap_hbm.at[b], g_v)
        base = b * CAP

        def step(i, _):
            rv = rank_v[pl.ds(i * 16, 16)]
            gv = g_v[pl.ds(i * 16, 16)]
            sel = gv < ZROW
            pos = gv - base
            toks = lax.iota(jnp.int32, 16) + i * 16
            plsc.store_scatter(sidx_v, [pos], toks, mask=sel)
            wvals = plsc.load_gather(rw2_v, [rv])
            plsc.store_scatter(wt_v, [pos], wvals, mask=sel)
            return 0

        lax.fori_loop(0, S // 16, step, 0)
        pltpu.sync_copy(wt_v, wts_hbm.at[b])
        pltpu.sync_copy(sidx_v, sidx_sh.at[s])

    plsc.subcore_barrier()

    half = s // WPB
    b = c * 2 + half
    w8 = s % WPB
    slot0 = w8 * (CAP // WPB)
    pltpu.sync_copy(sidx_sh.at[half, pl.ds(slot0, CAP // WPB)], idx_v)

    bufs = (rows_a, rows_b)
    gsems = (sem_a, sem_b)
    wsems = (wsem_a, wsem_b)
    xb = x_hbm.at[b]
    row00 = b * CAP + slot0

    def super_step(k, _):
        cps = []
        for q in range(2):
            ch = k * 2 + q

            @pl.when(k > 0)
            def _drain():
                pltpu.make_async_copy(
                    bufs[q], modx_hbm.at[pl.ds(row00, _GCH)], wsems[q]).wait()

            cps.append(pltpu.async_copy(
                xb.at[idx_v.at[pl.ds(ch * _GCH, _GCH)]], bufs[q], gsems[q]))
        for q in range(2):
            ch = k * 2 + q
            cps[q].wait()
            pltpu.async_copy(
                bufs[q], modx_hbm.at[pl.ds(row00 + ch * _GCH, _GCH)], wsems[q])
        return 0

    lax.fori_loop(0, _NCH // 2, super_step, 0)
    for q in range(2):
        pltpu.make_async_copy(
            bufs[q], modx_hbm.at[pl.ds(row00, _GCH)], wsems[q]).wait()


def _route_gather(rw2, rank, gmap, x):
    mesh = plsc.VectorSubcoreMesh(core_axis_name="c", subcore_axis_name="s",
                                  num_cores=NC, num_subcores=NS)
    f = pl.kernel(
        _route_gather_body,
        out_type=[
            jax.ShapeDtypeStruct((B, CAP), jnp.float32),
            jax.ShapeDtypeStruct((B * CAP, D), jnp.float32),
        ],
        mesh=mesh,
        scratch_types=[
            pltpu.VMEM((S,), jnp.float32),
            pltpu.VMEM((S,), jnp.int32),
            pltpu.VMEM((CAP,), jnp.int32),
            pltpu.VMEM((CAP,), jnp.float32),
            pltpu.VMEM((S,), jnp.int32),
            pltpu.VMEM((CAP // WPB,), jnp.int32),
            pltpu.VMEM((_GCH, D), jnp.float32),
            pltpu.VMEM((_GCH, D), jnp.float32),
            pltpu.VMEM_SHARED((2, CAP), jnp.int32),
            pltpu.SemaphoreType.DMA,
            pltpu.SemaphoreType.DMA,
            pltpu.SemaphoreType.DMA,
            pltpu.SemaphoreType.DMA,
        ],
        compiler_params=pltpu.CompilerParams(needs_layout_passes=False),
    )
    return f(rw2, rank, gmap, x)


_BM = 1024
_BK = 1024


_NCH_FFN = DFF // _BK


def _ffn_body(x_ref, w1_ref, w2_ref, wt_ref, out_ref):
    cch = pl.program_id(1)

    @pl.when(cch == 0)
    def _():
        out_ref[...] = jnp.zeros_like(out_ref)

    h = jax.nn.gelu(
        jnp.dot(x_ref[...].astype(jnp.bfloat16), w1_ref[...],
                preferred_element_type=jnp.float32))
    acc = out_ref[...] + jnp.dot(h.astype(jnp.bfloat16), w2_ref[...],
                                 preferred_element_type=jnp.float32)

    @pl.when(cch < _NCH_FFN - 1)
    def _():
        out_ref[...] = acc

    @pl.when(cch == _NCH_FFN - 1)
    def _():
        wt = wt_ref[0, 0, :]
        out_ref[...] = acc * wt[:, None]


def _ffn(modx, W1, W2, wts):
    w1b = W1.astype(jnp.bfloat16)
    w2b = W2.astype(jnp.bfloat16)
    wt2 = wts.reshape(B * CAP // _BM, 1, _BM)
    grid = (B * CAP // _BM, _NCH_FFN)
    out = pl.pallas_call(
        _ffn_body,
        grid=grid,
        in_specs=[
            pl.BlockSpec((_BM, D), lambda m, c: (m, 0)),
            pl.BlockSpec((D, _BK), lambda m, c: (0, c)),
            pl.BlockSpec((_BK, D), lambda m, c: (c, 0)),
            pl.BlockSpec((1, 1, _BM), lambda m, c: (m, 0, 0)),
        ],
        out_specs=pl.BlockSpec((_BM, D), lambda m, c: (m, 0)),
        out_shape=jax.ShapeDtypeStruct((B * CAP + 16, D), jnp.float32),
        scratch_shapes=[],
        compiler_params=pltpu.CompilerParams(
            dimension_semantics=("parallel", "arbitrary"),
            vmem_limit_bytes=63 * 1024 * 1024),
    )(modx, w1b, w2b, wt2)
    return out


_OCH = 16
_TPW = S // WPB


def _out_gather_body(gmap_hbm, modout_hbm, out_hbm,
                     g_v, zrow_v, rows_a, rows_b, sem_a, sem_b,
                     wsem_a, wsem_b):
    c = lax.axis_index("c")
    s = lax.axis_index("s")

    @pl.when(s == 0)
    def _zero_rows():
        def zb(i, _):
            zrow_v[pl.ds(i * 16, 16)] = jnp.zeros((16,), jnp.float32)
            return 0

        lax.fori_loop(0, D // 16, zb, 0)

        def zw(k, _):
            pltpu.sync_copy(zrow_v, modout_hbm.at[ZROW + k])
            return 0

        lax.fori_loop(0, 16, zw, 0)

    plsc.subcore_barrier()

    half = s // WPB
    b = c * 2 + half
    t0 = (s % WPB) * _TPW
    pltpu.sync_copy(gmap_hbm.at[b, pl.ds(t0, _TPW)], g_v)

    bufs = (rows_a, rows_b)
    gsems = (sem_a, sem_b)
    wsems = (wsem_a, wsem_b)
    ob = out_hbm.at[b]

    def super_step(k, _):
        cps = []
        for q in range(2):
            ch = k * 2 + q

            @pl.when(k > 0)
            def _drain():
                pltpu.make_async_copy(
                    bufs[q], ob.at[pl.ds(t0, _OCH)], wsems[q]).wait()

            cps.append(pltpu.async_copy(
                modout_hbm.at[g_v.at[pl.ds(ch * _OCH, _OCH)]], bufs[q],
                gsems[q]))
        for q in range(2):
            ch = k * 2 + q
            cps[q].wait()
            pltpu.async_copy(bufs[q], ob.at[pl.ds(t0 + ch * _OCH, _OCH)],
                             wsems[q])
        return 0

    lax.fori_loop(0, _TPW // _OCH // 2, super_step, 0)
    for q in range(2):
        pltpu.make_async_copy(
            bufs[q], ob.at[pl.ds(t0, _OCH)], wsems[q]).wait()


def _out_gather(gmap, modout):
    mesh = plsc.VectorSubcoreMesh(core_axis_name="c", subcore_axis_name="s",
                                  num_cores=NC, num_subcores=NS)
    f = pl.kernel(
        _out_gather_body,
        out_type=jax.ShapeDtypeStruct((B, S, D), jnp.float32),
        mesh=mesh,
        scratch_types=[
            pltpu.VMEM((_TPW,), jnp.int32),
            pltpu.VMEM((D,), jnp.float32),
            pltpu.VMEM((_OCH, D), jnp.float32),
            pltpu.VMEM((_OCH, D), jnp.float32),
            pltpu.SemaphoreType.DMA,
            pltpu.SemaphoreType.DMA,
            pltpu.SemaphoreType.DMA,
            pltpu.SemaphoreType.DMA,
        ],
        compiler_params=pltpu.CompilerParams(needs_layout_passes=False),
    )
    return f(gmap, modout)


def kernel(x, Wr, W1, W2):
    rw2_3d = _router(x, Wr)
    rank_3d, gmap, mod_loss = _ranks_loss(rw2_3d)
    wts, modx = _route_gather(
        rw2_3d.reshape(B, S), rank_3d.reshape(B, S), gmap, x)
    modout = _ffn(modx, W1, W2, wts)
    out = _out_gather(gmap, modout)
    return out, mod_loss
